# baseline (device time: 44416 ns/iter reference)
import jax
import jax.numpy as jnp
from jax import lax
from jax.experimental import pallas as pl
from jax.experimental.pallas import tpu as pltpu

M, N = 2048, 1024
Q = M // 4
NC = 4
CW = N // NC

BF16 = jnp.bfloat16


def kernel(x):
    def sem(phase, stream, c):
        return phase * (2 * NC) + stream * NC + c

    def body(x_ref, out_ref, ra1, rb1, ra2, rb2, ssems, rsems):
        mx = lax.axis_index("x")
        my = lax.axis_index("y")
        x_peer = (1 - mx, my)
        y_peer = (mx, 1 - my)

        a_mine = mx * Q
        a_theirs = (1 - mx) * Q
        b_mine = 2 * Q + my * Q
        b_theirs = 2 * Q + (1 - my) * Q

        def copy(src, dst, phase, stream, c, peer):
            return pltpu.make_async_remote_copy(
                src_ref=src, dst_ref=dst,
                send_sem=ssems.at[sem(phase, stream, c)],
                recv_sem=rsems.at[sem(phase, stream, c)],
                device_id=peer, device_id_type=pl.DeviceIdType.MESH,
            )

        barrier_sem = pltpu.get_barrier_semaphore()
        for nbr in (x_peer, y_peer):
            pl.semaphore_signal(
                barrier_sem, inc=1,
                device_id=nbr, device_id_type=pl.DeviceIdType.MESH,
            )
        pl.semaphore_wait(barrier_sem, 2)

        p1a, p1b = [], []
        for c in range(NC):
            cols = pl.ds(c * CW, CW)
            out_ref[pl.ds(a_theirs, Q), cols] = x_ref[
                0, 0, pl.ds(a_theirs, Q), cols].astype(BF16)
            d = copy(out_ref.at[pl.ds(a_theirs, Q), cols],
                     ra1.at[:, cols], 0, 0, c, x_peer)
            d.start()
            p1a.append(d)

            out_ref[pl.ds(b_theirs, Q), cols] = x_ref[
                0, 0, pl.ds(b_theirs, Q), cols].astype(BF16)
            d = copy(out_ref.at[pl.ds(b_theirs, Q), cols],
                     rb1.at[:, cols], 0, 1, c, y_peer)
            d.start()
            p1b.append(d)

        out_ref[pl.ds(a_mine, Q), :] = x_ref[
            0, 0, pl.ds(a_mine, Q), :].astype(BF16)
        out_ref[pl.ds(b_mine, Q), :] = x_ref[
            0, 0, pl.ds(b_mine, Q), :].astype(BF16)

        p2a, p2b = [], []
        for c in range(NC):
            cols = pl.ds(c * CW, CW)
            p1a[c].wait_recv()
            out_ref[pl.ds(a_mine, Q), cols] = (
                out_ref[pl.ds(a_mine, Q), cols] + ra1[:, cols]
            )
            d = copy(out_ref.at[pl.ds(a_mine, Q), cols],
                     ra2.at[:, cols], 1, 0, c, y_peer)
            d.start()
            p2a.append(d)

            p1b[c].wait_recv()
            out_ref[pl.ds(b_mine, Q), cols] = (
                out_ref[pl.ds(b_mine, Q), cols] + rb1[:, cols]
            )
            d = copy(out_ref.at[pl.ds(b_mine, Q), cols],
                     rb2.at[:, cols], 1, 1, c, x_peer)
            d.start()
            p2b.append(d)

        p3a, p3b = [], []
        for c in range(NC):
            cols = pl.ds(c * CW, CW)
            p2a[c].wait()
            out_ref[pl.ds(a_mine, Q), cols] = (
                out_ref[pl.ds(a_mine, Q), cols] + ra2[:, cols]
            )
            d = copy(out_ref.at[pl.ds(a_mine, Q), cols],
                     out_ref.at[pl.ds(a_mine, Q), cols], 2, 0, c, x_peer)
            d.start()
            p3a.append(d)

            p2b[c].wait()
            out_ref[pl.ds(b_mine, Q), cols] = (
                out_ref[pl.ds(b_mine, Q), cols] + rb2[:, cols]
            )
            d = copy(out_ref.at[pl.ds(b_mine, Q), cols],
                     out_ref.at[pl.ds(b_mine, Q), cols], 2, 1, c, y_peer)
            d.start()
            p3b.append(d)

        for c in range(NC):
            cols = pl.ds(c * CW, CW)
            copy(out_ref.at[pl.ds(a_mine, Q), cols],
                 out_ref.at[pl.ds(a_theirs, Q), cols],
                 2, 0, c, x_peer).wait_recv()
            copy(out_ref.at[pl.ds(b_mine, Q), cols],
                 out_ref.at[pl.ds(b_theirs, Q), cols],
                 2, 1, c, y_peer).wait_recv()

        for d in p1a + p1b + p3a + p3b:
            d.wait_send()

    return pl.pallas_call(
        body,
        out_shape=jax.ShapeDtypeStruct((M, N), BF16),
        in_specs=[pl.BlockSpec(memory_space=pltpu.VMEM)],
        out_specs=pl.BlockSpec(memory_space=pltpu.VMEM),
        scratch_shapes=[
            pltpu.VMEM((Q, N), BF16),
            pltpu.VMEM((Q, N), BF16),
            pltpu.VMEM((Q, N), BF16),
            pltpu.VMEM((Q, N), BF16),
            pltpu.SemaphoreType.DMA((3 * 2 * NC,)),
            pltpu.SemaphoreType.DMA((3 * 2 * NC,)),
        ],
        compiler_params=pltpu.CompilerParams(collective_id=0),
    )(x)


# device time: 44378 ns/iter; 1.0009x vs baseline; 1.0009x over previous
import jax
import jax.numpy as jnp
from jax import lax
from jax.experimental import pallas as pl
from jax.experimental.pallas import tpu as pltpu

M, N = 2048, 1024
Q = M // 4
NC = 4
QC = Q // NC

BF16 = jnp.bfloat16


def kernel(x):
    def sem(phase, stream, c):
        return phase * (2 * NC) + stream * NC + c

    def body(x_ref, out_ref, ra1, rb1, ra2, rb2, ssems, rsems):
        mx = lax.axis_index("x")
        my = lax.axis_index("y")
        x_peer = (1 - mx, my)
        y_peer = (mx, 1 - my)

        a_mine = mx * Q
        a_theirs = (1 - mx) * Q
        b_mine = 2 * Q + my * Q
        b_theirs = 2 * Q + (1 - my) * Q

        def copy(src, dst, phase, stream, c, peer):
            return pltpu.make_async_remote_copy(
                src_ref=src, dst_ref=dst,
                send_sem=ssems.at[sem(phase, stream, c)],
                recv_sem=rsems.at[sem(phase, stream, c)],
                device_id=peer, device_id_type=pl.DeviceIdType.MESH,
            )

        barrier_sem = pltpu.get_barrier_semaphore()
        for nbr in (x_peer, y_peer):
            pl.semaphore_signal(
                barrier_sem, inc=1,
                device_id=nbr, device_id_type=pl.DeviceIdType.MESH,
            )
        pl.semaphore_wait(barrier_sem, 2)

        p1a, p1b = [], []
        for c in range(NC):
            rows = pl.ds(a_theirs + c * QC, QC)
            out_ref[rows, :] = x_ref[0, 0, rows, :].astype(BF16)
            d = copy(out_ref.at[rows], ra1.at[pl.ds(c * QC, QC)],
                     0, 0, c, x_peer)
            d.start()
            p1a.append(d)

            rows = pl.ds(b_theirs + c * QC, QC)
            out_ref[rows, :] = x_ref[0, 0, rows, :].astype(BF16)
            d = copy(out_ref.at[rows], rb1.at[pl.ds(c * QC, QC)],
                     0, 1, c, y_peer)
            d.start()
            p1b.append(d)

        out_ref[pl.ds(a_mine, Q), :] = x_ref[
            0, 0, pl.ds(a_mine, Q), :].astype(BF16)
        out_ref[pl.ds(b_mine, Q), :] = x_ref[
            0, 0, pl.ds(b_mine, Q), :].astype(BF16)

        p2a, p2b = [], []
        for c in range(NC):
            ch = pl.ds(c * QC, QC)

            p1a[c].wait_recv()
            rows = pl.ds(a_mine + c * QC, QC)
            out_ref[rows, :] = out_ref[rows, :] + ra1[ch, :]
            d = copy(out_ref.at[rows], ra2.at[ch], 1, 0, c, y_peer)
            d.start()
            p2a.append(d)

            p1b[c].wait_recv()
            rows = pl.ds(b_mine + c * QC, QC)
            out_ref[rows, :] = out_ref[rows, :] + rb1[ch, :]
            d = copy(out_ref.at[rows], rb2.at[ch], 1, 1, c, x_peer)
            d.start()
            p2b.append(d)

        p3a, p3b = [], []
        for c in range(NC):
            ch = pl.ds(c * QC, QC)

            p2a[c].wait()
            rows = pl.ds(a_mine + c * QC, QC)
            out_ref[rows, :] = out_ref[rows, :] + ra2[ch, :]
            d = copy(out_ref.at[rows], out_ref.at[rows], 2, 0, c, x_peer)
            d.start()
            p3a.append(d)

            p2b[c].wait()
            rows = pl.ds(b_mine + c * QC, QC)
            out_ref[rows, :] = out_ref[rows, :] + rb2[ch, :]
            d = copy(out_ref.at[rows], out_ref.at[rows], 2, 1, c, y_peer)
            d.start()
            p3b.append(d)

        for c in range(NC):
            copy(out_ref.at[pl.ds(a_mine + c * QC, QC)],
                 out_ref.at[pl.ds(a_theirs + c * QC, QC)],
                 2, 0, c, x_peer).wait_recv()
            copy(out_ref.at[pl.ds(b_mine + c * QC, QC)],
                 out_ref.at[pl.ds(b_theirs + c * QC, QC)],
                 2, 1, c, y_peer).wait_recv()

        for d in p1a + p1b + p3a + p3b:
            d.wait_send()

    return pl.pallas_call(
        body,
        out_shape=jax.ShapeDtypeStruct((M, N), BF16),
        in_specs=[pl.BlockSpec(memory_space=pltpu.VMEM)],
        out_specs=pl.BlockSpec(memory_space=pltpu.VMEM),
        scratch_shapes=[
            pltpu.VMEM((Q, N), BF16),
            pltpu.VMEM((Q, N), BF16),
            pltpu.VMEM((Q, N), BF16),
            pltpu.VMEM((Q, N), BF16),
            pltpu.SemaphoreType.DMA((3 * 2 * NC,)),
            pltpu.SemaphoreType.DMA((3 * 2 * NC,)),
        ],
        compiler_params=pltpu.CompilerParams(collective_id=0),
    )(x)
